# baseline (speedup 1.0000x reference)
import functools

import jax
import jax.numpy as jnp
from jax import lax
from jax.experimental import pallas as pl
from jax.experimental.pallas import tpu as pltpu
from jax.experimental.pallas import tpu_sc as plsc

DIM = 64
CHUNK = 128


@functools.cache
def _build(n_ids: int, vocab: int, dim: int):
    info = plsc.get_sparse_core_info()
    nw = info.num_cores * info.num_subcores
    assert n_ids % (nw * CHUNK) == 0
    per_w = n_ids // nw
    n_chunks = per_w // CHUNK

    mesh = plsc.VectorSubcoreMesh(core_axis_name="c", subcore_axis_name="s")

    @functools.partial(
        pl.kernel,
        mesh=mesh,
        out_type=jax.ShapeDtypeStruct((n_ids, dim), jnp.float32),
        scratch_types=[
            pltpu.VMEM((n_chunks, CHUNK), jnp.int32),
            pltpu.VMEM((CHUNK, dim), jnp.float32),
            pltpu.SemaphoreType.DMA,
        ],
        compiler_params=pltpu.CompilerParams(use_tc_tiling_on_sc=False),
    )
    def gather_kernel(ids_hbm, table_hbm, out_hbm, idx_v, rows_v, gsem):
        wid = lax.axis_index("s") * info.num_cores + lax.axis_index("c")
        base = wid * per_w
        pltpu.sync_copy(ids_hbm.at[wid], idx_v)

        def step(g, carry):
            pltpu.async_copy(table_hbm.at[idx_v.at[g]], rows_v, gsem).wait()
            pltpu.sync_copy(rows_v, out_hbm.at[pl.ds(base + g * CHUNK, CHUNK)])
            return carry

        lax.fori_loop(0, n_chunks, step, 0)

    def run(ids_flat, table):
        ids3 = ids_flat.reshape(nw, n_chunks, CHUNK)
        return gather_kernel(ids3, table)

    return run


def kernel(ids, emb_matrix):
    b, t = ids.shape
    vocab, dim = emb_matrix.shape
    flat = ids.reshape(b * t).astype(jnp.int32)
    out = _build(b * t, vocab, dim)(flat, emb_matrix)
    return out.reshape(b, t, dim)


# speedup vs baseline: 1.0602x; 1.0602x over previous
import functools

import jax
import jax.numpy as jnp
from jax import lax
from jax.experimental import pallas as pl
from jax.experimental.pallas import tpu as pltpu
from jax.experimental.pallas import tpu_sc as plsc

DIM = 64
CHUNK = 128


GROUP = 4
NBUF = 2


@functools.cache
def _build(n_ids: int, vocab: int, dim: int):
    info = plsc.get_sparse_core_info()
    nw = info.num_cores * info.num_subcores
    assert n_ids % (nw * CHUNK * GROUP * NBUF) == 0
    per_w = n_ids // nw
    n_chunks = per_w // CHUNK
    n_groups = n_chunks // GROUP
    n_outer = n_groups // NBUF
    grows = GROUP * CHUNK

    mesh = plsc.VectorSubcoreMesh(core_axis_name="c", subcore_axis_name="s")

    @functools.partial(
        pl.kernel,
        mesh=mesh,
        out_type=jax.ShapeDtypeStruct((n_ids, dim), jnp.float32),
        scratch_types=[
            pltpu.VMEM((n_chunks, CHUNK), jnp.int32),
            pltpu.VMEM((NBUF, grows, dim), jnp.float32),
            [pltpu.SemaphoreType.DMA] * NBUF,
        ],
        compiler_params=pltpu.CompilerParams(use_tc_tiling_on_sc=False),
    )
    def gather_kernel(ids_hbm, table_hbm, out_hbm, idx_v, rows_v, gsems):
        wid = lax.axis_index("s") * info.num_cores + lax.axis_index("c")
        base = wid * per_w
        pltpu.sync_copy(ids_hbm.at[wid], idx_v)

        def fire(g, b):
            for k in range(GROUP):
                pltpu.async_copy(
                    table_hbm.at[idx_v.at[g * GROUP + k]],
                    rows_v.at[b].at[pl.ds(k * CHUNK, CHUNK)],
                    gsems[b],
                )

        def drain_write(g, b):
            for k in range(GROUP):
                pltpu.make_async_copy(
                    table_hbm.at[idx_v.at[g * GROUP + k]],
                    rows_v.at[b].at[pl.ds(k * CHUNK, CHUNK)],
                    gsems[b],
                ).wait()
            pltpu.sync_copy(rows_v.at[b], out_hbm.at[pl.ds(base + g * grows, grows)])

        for b in range(NBUF):
            fire(b, b)

        def outer(i, carry):
            for b in range(NBUF):
                g = i * NBUF + b
                drain_write(g, b)
                fire(g + NBUF, b)
            return carry

        lax.fori_loop(0, n_outer - 1, outer, 0)
        for b in range(NBUF):
            drain_write((n_outer - 1) * NBUF + b, b)

    def run(ids_flat, table):
        ids3 = ids_flat.reshape(nw, n_chunks, CHUNK)
        return gather_kernel(ids3, table)

    return run


def kernel(ids, emb_matrix):
    b, t = ids.shape
    vocab, dim = emb_matrix.shape
    flat = ids.reshape(b * t).astype(jnp.int32)
    out = _build(b * t, vocab, dim)(flat, emb_matrix)
    return out.reshape(b, t, dim)


# speedup vs baseline: 1.1069x; 1.0440x over previous
import functools

import jax
import jax.numpy as jnp
from jax import lax
from jax.experimental import pallas as pl
from jax.experimental.pallas import tpu as pltpu
from jax.experimental.pallas import tpu_sc as plsc

DIM = 64
CHUNK = 128
NBUF = 4


@functools.cache
def _build(n_b: int, n_t: int, vocab: int):
    info = plsc.get_sparse_core_info()
    nc = info.num_cores
    nw = nc * info.num_subcores
    b_per_w = n_b // nw
    cpt = b_per_w // CHUNK
    n_chunks = n_t * cpt
    n_outer = n_chunks // NBUF
    assert b_per_w % CHUNK == 0 and n_chunks % NBUF == 0

    mesh = plsc.VectorSubcoreMesh(core_axis_name="c", subcore_axis_name="s")

    @functools.partial(
        pl.kernel,
        mesh=mesh,
        out_type=jax.ShapeDtypeStruct((n_t, n_b, DIM), jnp.float32),
        scratch_types=[
            pltpu.VMEM((n_t, b_per_w), jnp.int32),
            pltpu.VMEM((NBUF, CHUNK, DIM), jnp.float32),
            [pltpu.SemaphoreType.DMA] * NBUF,
        ],
        compiler_params=pltpu.CompilerParams(use_tc_tiling_on_sc=False),
    )
    def gather_kernel(ids_hbm, table_hbm, out_hbm, idx_v, rows_v, gsems):
        wid = lax.axis_index("s") * nc + lax.axis_index("c")
        b0 = wid * b_per_w
        pltpu.sync_copy(ids_hbm.at[:, pl.ds(b0, b_per_w)], idx_v)

        def fire(g, b):
            t = g // cpt
            j = g - t * cpt
            pltpu.async_copy(
                table_hbm.at[idx_v.at[t].at[pl.ds(j * CHUNK, CHUNK)]],
                rows_v.at[b],
                gsems[b],
            )

        def drain_write(g, b):
            t = g // cpt
            j = g - t * cpt
            pltpu.make_async_copy(
                table_hbm.at[idx_v.at[t].at[pl.ds(j * CHUNK, CHUNK)]],
                rows_v.at[b],
                gsems[b],
            ).wait()
            pltpu.sync_copy(
                rows_v.at[b],
                out_hbm.at[t].at[pl.ds(b0 + j * CHUNK, CHUNK)],
            )

        for b in range(NBUF):
            fire(b, b)

        def outer(i, carry):
            for b in range(NBUF):
                g = i * NBUF + b
                drain_write(g, b)
                fire(g + NBUF, b)
            return carry

        lax.fori_loop(0, n_outer - 1, outer, 0)
        for b in range(NBUF):
            drain_write((n_outer - 1) * NBUF + b, b)

    return gather_kernel


def kernel(ids, emb_matrix):
    n_b, n_t = ids.shape
    vocab, dim = emb_matrix.shape
    ids_t = jnp.swapaxes(ids, 0, 1).astype(jnp.int32)
    out3 = _build(n_b, n_t, vocab)(ids_t, emb_matrix)
    return jnp.swapaxes(out3, 0, 1)
